# baseline (speedup 1.0000x reference)
import functools

import jax
import jax.numpy as jnp
from jax import lax
from jax.experimental import pallas as pl
from jax.experimental.pallas import tpu as pltpu
from jax.experimental.pallas import tpu_sc as plsc

BATCH = 4
SEQ = 8192
EMB = 1024
LANES = 16
WORKERS = 32
WPR = WORKERS // BATCH
CHUNK = SEQ // WPR

_mesh = plsc.VectorSubcoreMesh(core_axis_name="c", subcore_axis_name="s")


@functools.partial(
    pl.kernel,
    mesh=_mesh,
    out_type=(
        jax.ShapeDtypeStruct((BATCH, EMB), jnp.float32),
        jax.ShapeDtypeStruct((BATCH, SEQ), jnp.float32),
    ),
    scratch_types=[
        pltpu.VMEM((SEQ,), jnp.int32),
        pltpu.VMEM((CHUNK,), jnp.float32),
        pltpu.VMEM((EMB,), jnp.float32),
    ],
    compiler_params=pltpu.CompilerParams(needs_layout_passes=False),
)
def _last_pool_sc(x_hbm, mask_hbm, ctx_hbm, w_hbm, mvec, wbuf, ctxbuf):
    wid = lax.axis_index("c") * 16 + lax.axis_index("s")
    row = wid // WPR
    chunk = wid % WPR
    base = chunk * CHUNK

    pltpu.sync_copy(mask_hbm.at[row], mvec)

    def sum_body(i, acc):
        return acc + mvec[pl.ds(i * LANES, LANES)]

    acc = lax.fori_loop(0, SEQ // LANES, sum_body,
                        jnp.zeros((LANES,), jnp.int32))
    length = jnp.sum(acc)
    idx = jnp.maximum(length - 1, 0)

    iota = lax.iota(jnp.int32, LANES)

    def w_body(j, carry):
        pos = base + j * LANES + iota
        one = (pos == idx).astype(jnp.float32)
        wbuf[pl.ds(j * LANES, LANES)] = one
        return carry

    lax.fori_loop(0, CHUNK // LANES, w_body, 0)
    pltpu.sync_copy(wbuf, w_hbm.at[row, pl.ds(base, CHUNK)])

    @pl.when(chunk == 0)
    def _():
        pltpu.sync_copy(x_hbm.at[row, idx], ctxbuf)
        pltpu.sync_copy(ctxbuf, ctx_hbm.at[row])


def kernel(x, padding_mask):
    mask_i32 = padding_mask.astype(jnp.int32)
    return _last_pool_sc(x, mask_i32)


# speedup vs baseline: 4.8410x; 4.8410x over previous
import functools

import jax
import jax.numpy as jnp
from jax import lax
from jax.experimental import pallas as pl
from jax.experimental.pallas import tpu as pltpu

BATCH = 4
SEQ = 8192
EMB = 1024


def _body(mask_ref, x_ref, ctx_ref, w_ref, idx_vmem, idx_smem, sem, dma_sems):
    m = mask_ref[...].astype(jnp.int32)
    lengths = jnp.sum(m, axis=1)
    idx = jnp.maximum(lengths - 1, 0)

    iota = lax.broadcasted_iota(jnp.int32, (BATCH, SEQ), 1)
    w_ref[...] = (iota == idx[:, None]).astype(jnp.float32)

    idx_vmem[...] = idx
    pltpu.make_async_copy(idx_vmem, idx_smem, sem).start()
    pltpu.make_async_copy(idx_vmem, idx_smem, sem).wait()

    for b in range(BATCH):
        ib = idx_smem[b]
        pltpu.make_async_copy(
            x_ref.at[b, ib], ctx_ref.at[b], dma_sems.at[b]
        ).start()
    for b in range(BATCH):
        pltpu.make_async_copy(
            x_ref.at[b, idx_smem[b]], ctx_ref.at[b], dma_sems.at[b]
        ).wait()


@functools.partial(jax.jit, donate_argnums=())
def _last_pool(x, padding_mask):
    return pl.pallas_call(
        _body,
        grid=(1,),
        in_specs=[
            pl.BlockSpec((BATCH, SEQ), lambda i: (0, 0)),
            pl.BlockSpec(memory_space=pl.ANY),
        ],
        out_specs=[
            pl.BlockSpec((BATCH, EMB), lambda i: (0, 0)),
            pl.BlockSpec((BATCH, SEQ), lambda i: (0, 0)),
        ],
        out_shape=[
            jax.ShapeDtypeStruct((BATCH, EMB), jnp.float32),
            jax.ShapeDtypeStruct((BATCH, SEQ), jnp.float32),
        ],
        scratch_shapes=[
            pltpu.VMEM((BATCH,), jnp.int32),
            pltpu.SMEM((BATCH,), jnp.int32),
            pltpu.SemaphoreType.DMA,
            pltpu.SemaphoreType.DMA((BATCH,)),
        ],
    )(padding_mask, x)


def kernel(x, padding_mask):
    ctx, w = _last_pool(x, padding_mask)
    return (ctx, w)
